# Initial kernel scaffold; baseline (speedup 1.0000x reference)
import jax
import jax.numpy as jnp
from jax.experimental import pallas as pl


def kernel(loc_data, conf_data, priors, boxes, labels):
    raise NotImplementedError("write your pallas kernel here")


# speedup vs baseline: 6.6294x; 6.6294x over previous
import functools

import jax
import jax.numpy as jnp
from jax.experimental import pallas as pl
from jax.experimental.pallas import tpu as pltpu

_NUM_CLASSES = 81
_THRESHOLD = 0.5
_NEGPOS_RATIO = 3
_V0, _V1 = 0.1, 0.2

_P = 24564
_PPAD = 24576
_ROWS, _LANES = 192, 128
_NOBJ = 20
_TP = 2048
_NT = _PPAD // _TP


def _match_kernel(priors_ref, boxes_ref, labels_ref, loc_ref,
                  conf_t_ref, num_pos_ref, loss_l_ref):
    pcx = priors_ref[0]
    pcy = priors_ref[1]
    pw = priors_ref[2]
    ph = priors_ref[3]
    px1 = pcx - pw * 0.5
    py1 = pcy - ph * 0.5
    px2 = pcx + pw * 0.5
    py2 = pcy + ph * 0.5
    area_p = pw * ph

    lin = (jax.lax.broadcasted_iota(jnp.int32, (_ROWS, _LANES), 0) * _LANES
           + jax.lax.broadcasted_iota(jnp.int32, (_ROWS, _LANES), 1))

    bto = jnp.full((_ROWS, _LANES), -1.0, dtype=jnp.float32)
    bti = jnp.zeros((_ROWS, _LANES), dtype=jnp.int32)
    bpi = []
    for t in range(_NOBJ):
        bx1 = boxes_ref[0, t, 0]
        by1 = boxes_ref[0, t, 1]
        bx2 = boxes_ref[0, t, 2]
        by2 = boxes_ref[0, t, 3]
        area_t = (bx2 - bx1) * (by2 - by1)
        ix = jnp.maximum(jnp.minimum(px2, bx2) - jnp.maximum(px1, bx1), 0.0)
        iy = jnp.maximum(jnp.minimum(py2, by2) - jnp.maximum(py1, by1), 0.0)
        inter = ix * iy
        ov = inter / (area_t + area_p - inter)
        upd = ov > bto
        bto = jnp.where(upd, ov, bto)
        bti = jnp.where(upd, t, bti)
        m = jnp.max(ov)
        cand = jnp.where(ov == m, lin, jnp.int32(0x7FFFFFFF))
        bpi.append(jnp.min(cand))
    for t in range(_NOBJ):
        hit = lin == bpi[t]
        bto = jnp.where(hit, 2.0, bto)
        bti = jnp.where(hit, t, bti)

    conf = jnp.zeros((_ROWS, _LANES), dtype=jnp.int32)
    mx1 = jnp.zeros((_ROWS, _LANES), dtype=jnp.float32)
    my1 = jnp.zeros((_ROWS, _LANES), dtype=jnp.float32)
    mx2 = jnp.zeros((_ROWS, _LANES), dtype=jnp.float32)
    my2 = jnp.zeros((_ROWS, _LANES), dtype=jnp.float32)
    for t in range(_NOBJ):
        sel = bti == t
        conf = jnp.where(sel, labels_ref[0, 0, t] + 1, conf)
        mx1 = jnp.where(sel, boxes_ref[0, t, 0], mx1)
        my1 = jnp.where(sel, boxes_ref[0, t, 1], my1)
        mx2 = jnp.where(sel, boxes_ref[0, t, 2], mx2)
        my2 = jnp.where(sel, boxes_ref[0, t, 3], my2)
    conf = jnp.where(bto < _THRESHOLD, 0, conf)
    pos = conf > 0
    conf_t_ref[0] = conf
    num_pos_ref[0, 0, 0] = jnp.sum(pos.astype(jnp.int32))

    pws = jnp.where(pw > 0.0, pw, 1.0)
    phs = jnp.where(ph > 0.0, ph, 1.0)
    g_cx = ((mx1 + mx2) * 0.5 - pcx) / (_V0 * pws)
    g_cy = ((my1 + my2) * 0.5 - pcy) / (_V0 * phs)
    g_w = jnp.log((mx2 - mx1) / pws) / _V1
    g_h = jnp.log((my2 - my1) / phs) / _V1
    acc = jnp.zeros((_ROWS, _LANES), dtype=jnp.float32)
    for i, g in enumerate((g_cx, g_cy, g_w, g_h)):
        d = loc_ref[0, i] - g
        a = jnp.abs(d)
        sl1 = jnp.where(a < 1.0, 0.5 * d * d, a - 0.5)
        acc = acc + jnp.where(pos, sl1, 0.0)
    loss_l_ref[0, 0, 0] = jnp.sum(acc)


def _conf_kernel(conf_ref, conf_t_ref, ce_ref, pos_sum_ref):
    j = pl.program_id(1)
    x = conf_ref[0]
    m = jnp.max(x, axis=1, keepdims=True)
    lse = m + jnp.log(jnp.sum(jnp.exp(x - m), axis=1, keepdims=True))
    cls = jax.lax.broadcasted_iota(jnp.int32, (_TP, _NUM_CLASSES), 1)
    tt = conf_t_ref[0]
    gathered = jnp.sum(jnp.where(cls == tt, x, 0.0), axis=1, keepdims=True)
    ce = lse - gathered
    pos = tt > 0
    pidx = jax.lax.broadcasted_iota(jnp.int32, (_TP, 1), 0) + j * _TP
    valid = pidx < _P
    ce_ref[0] = jnp.where(valid & jnp.logical_not(pos), ce, 0.0)
    psum = jnp.sum(jnp.where(valid & pos, ce, 0.0))

    @pl.when(j == 0)
    def _():
        pos_sum_ref[0, 0, 0] = psum

    @pl.when(j > 0)
    def _():
        pos_sum_ref[0, 0, 0] += psum


def _mine_kernel(ce_ref, num_pos_ref, pos_sum_ref, loss_l_ref,
                 out_l_ref, out_c_ref):
    ce = ce_ref[...]
    num = ce.shape[0]
    np_i = num_pos_ref[...]
    k = jnp.minimum(np_i * _NEGPOS_RATIO, _P - 1).astype(jnp.float32)

    def body(i, tbits):
        bit = jnp.left_shift(jnp.int32(1), 30 - i)
        trial = jnp.bitwise_or(tbits, bit)
        trialf = jax.lax.bitcast_convert_type(trial, jnp.float32)
        cnt = jnp.sum((ce >= trialf).astype(jnp.float32), axis=1,
                      keepdims=True)
        return jnp.where(cnt >= k, trial, tbits)

    tbits = jax.lax.fori_loop(0, 31, body,
                              jnp.zeros((num, 1), dtype=jnp.int32))
    thr = jax.lax.bitcast_convert_type(tbits, jnp.float32)
    gt = ce > thr
    sum_gt = jnp.sum(jnp.where(gt, ce, 0.0), axis=1, keepdims=True)
    cnt_gt = jnp.sum(gt.astype(jnp.float32), axis=1, keepdims=True)
    top_sum = sum_gt + (k - cnt_gt) * thr
    loss_c = jnp.sum(pos_sum_ref[...] + top_sum)
    n = jnp.sum(np_i).astype(jnp.float32)
    out_l_ref[0, 0] = jnp.sum(loss_l_ref[...]) / n
    out_c_ref[0, 0] = loss_c / n


def kernel(loc_data, conf_data, priors, boxes, labels):
    num = conf_data.shape[0]
    priors_p = jnp.pad(priors, ((0, _PPAD - _P), (0, 0))).T.reshape(
        4, _ROWS, _LANES)
    loc_p = jnp.pad(loc_data, ((0, 0), (0, _PPAD - _P), (0, 0))).transpose(
        0, 2, 1).reshape(num, 4, _ROWS, _LANES)
    labels3 = labels.reshape(num, 1, _NOBJ)

    conf_t, num_pos, loss_l_b = pl.pallas_call(
        _match_kernel,
        grid=(num,),
        in_specs=[
            pl.BlockSpec((4, _ROWS, _LANES), lambda b: (0, 0, 0)),
            pl.BlockSpec((1, _NOBJ, 4), lambda b: (b, 0, 0),
                         memory_space=pltpu.SMEM),
            pl.BlockSpec((1, 1, _NOBJ), lambda b: (b, 0, 0),
                         memory_space=pltpu.SMEM),
            pl.BlockSpec((1, 4, _ROWS, _LANES), lambda b: (b, 0, 0, 0)),
        ],
        out_specs=[
            pl.BlockSpec((1, _ROWS, _LANES), lambda b: (b, 0, 0)),
            pl.BlockSpec((1, 1, 1), lambda b: (b, 0, 0),
                         memory_space=pltpu.SMEM),
            pl.BlockSpec((1, 1, 1), lambda b: (b, 0, 0),
                         memory_space=pltpu.SMEM),
        ],
        out_shape=[
            jax.ShapeDtypeStruct((num, _ROWS, _LANES), jnp.int32),
            jax.ShapeDtypeStruct((num, 1, 1), jnp.int32),
            jax.ShapeDtypeStruct((num, 1, 1), jnp.float32),
        ],
    )(priors_p, boxes, labels3, loc_p)

    conf_t_col = conf_t.reshape(num * _NT, _TP, 1)

    ce, pos_sum = pl.pallas_call(
        _conf_kernel,
        grid=(num, _NT),
        in_specs=[
            pl.BlockSpec((1, _TP, _NUM_CLASSES), lambda b, j: (b, j, 0)),
            pl.BlockSpec((1, _TP, 1), lambda b, j: (b * _NT + j, 0, 0)),
        ],
        out_specs=[
            pl.BlockSpec((1, _TP, 1), lambda b, j: (b * _NT + j, 0, 0)),
            pl.BlockSpec((1, 1, 1), lambda b, j: (b, 0, 0),
                         memory_space=pltpu.SMEM),
        ],
        out_shape=[
            jax.ShapeDtypeStruct((num * _NT, _TP, 1), jnp.float32),
            jax.ShapeDtypeStruct((num, 1, 1), jnp.float32),
        ],
    )(conf_data, conf_t_col)

    ce2 = ce.reshape(num, _PPAD)

    out_l, out_c = pl.pallas_call(
        _mine_kernel,
        in_specs=[
            pl.BlockSpec(memory_space=pltpu.VMEM),
            pl.BlockSpec(memory_space=pltpu.VMEM),
            pl.BlockSpec(memory_space=pltpu.VMEM),
            pl.BlockSpec(memory_space=pltpu.VMEM),
        ],
        out_specs=[
            pl.BlockSpec(memory_space=pltpu.SMEM),
            pl.BlockSpec(memory_space=pltpu.SMEM),
        ],
        out_shape=[
            jax.ShapeDtypeStruct((1, 1), jnp.float32),
            jax.ShapeDtypeStruct((1, 1), jnp.float32),
        ],
    )(ce2, num_pos.reshape(num, 1), pos_sum.reshape(num, 1),
      loss_l_b.reshape(num, 1))

    return out_l.reshape(()), out_c.reshape(())


# speedup vs baseline: 13.2386x; 1.9969x over previous
import functools

import jax
import jax.numpy as jnp
from jax.experimental import pallas as pl
from jax.experimental.pallas import tpu as pltpu

_NUM_CLASSES = 81
_THRESHOLD = 0.5
_NEGPOS_RATIO = 3
_V0, _V1 = 0.1, 0.2

_P = 24564
_PPAD = 24576
_ROWS, _LANES = 192, 128
_NOBJ = 20
_TP = 1024
_NT = _PPAD // _TP


def _match_kernel(priors_ref, boxes_ref, labels_ref, loc_ref,
                  conf_t_ref, num_pos_ref, loss_l_ref):
    pcx = priors_ref[0]
    pcy = priors_ref[1]
    pw = priors_ref[2]
    ph = priors_ref[3]
    px1 = pcx - pw * 0.5
    py1 = pcy - ph * 0.5
    px2 = pcx + pw * 0.5
    py2 = pcy + ph * 0.5
    area_p = pw * ph

    lin = (jax.lax.broadcasted_iota(jnp.int32, (_ROWS, _LANES), 0) * _LANES
           + jax.lax.broadcasted_iota(jnp.int32, (_ROWS, _LANES), 1))

    bto = jnp.full((_ROWS, _LANES), -1.0, dtype=jnp.float32)
    bti = jnp.zeros((_ROWS, _LANES), dtype=jnp.int32)
    bpi = []
    for t in range(_NOBJ):
        bx1 = boxes_ref[0, t, 0]
        by1 = boxes_ref[0, t, 1]
        bx2 = boxes_ref[0, t, 2]
        by2 = boxes_ref[0, t, 3]
        area_t = (bx2 - bx1) * (by2 - by1)
        ix = jnp.maximum(jnp.minimum(px2, bx2) - jnp.maximum(px1, bx1), 0.0)
        iy = jnp.maximum(jnp.minimum(py2, by2) - jnp.maximum(py1, by1), 0.0)
        inter = ix * iy
        ov = inter / (area_t + area_p - inter)
        upd = ov > bto
        bto = jnp.where(upd, ov, bto)
        bti = jnp.where(upd, t, bti)
        m = jnp.max(ov)
        cand = jnp.where(ov == m, lin, jnp.int32(0x7FFFFFFF))
        bpi.append(jnp.min(cand))
    for t in range(_NOBJ):
        hit = lin == bpi[t]
        bto = jnp.where(hit, 2.0, bto)
        bti = jnp.where(hit, t, bti)

    conf = jnp.zeros((_ROWS, _LANES), dtype=jnp.int32)
    mx1 = jnp.zeros((_ROWS, _LANES), dtype=jnp.float32)
    my1 = jnp.zeros((_ROWS, _LANES), dtype=jnp.float32)
    mx2 = jnp.zeros((_ROWS, _LANES), dtype=jnp.float32)
    my2 = jnp.zeros((_ROWS, _LANES), dtype=jnp.float32)
    for t in range(_NOBJ):
        sel = bti == t
        conf = jnp.where(sel, labels_ref[0, 0, t] + 1, conf)
        mx1 = jnp.where(sel, boxes_ref[0, t, 0], mx1)
        my1 = jnp.where(sel, boxes_ref[0, t, 1], my1)
        mx2 = jnp.where(sel, boxes_ref[0, t, 2], mx2)
        my2 = jnp.where(sel, boxes_ref[0, t, 3], my2)
    conf = jnp.where(bto < _THRESHOLD, 0, conf)
    pos = conf > 0
    conf_t_ref[0] = conf
    num_pos_ref[0, 0, 0] = jnp.sum(pos.astype(jnp.int32))

    pws = jnp.where(pw > 0.0, pw, 1.0)
    phs = jnp.where(ph > 0.0, ph, 1.0)
    g_cx = ((mx1 + mx2) * 0.5 - pcx) / (_V0 * pws)
    g_cy = ((my1 + my2) * 0.5 - pcy) / (_V0 * phs)
    g_w = jnp.log((mx2 - mx1) / pws) / _V1
    g_h = jnp.log((my2 - my1) / phs) / _V1
    acc = jnp.zeros((_ROWS, _LANES), dtype=jnp.float32)
    for i, g in enumerate((g_cx, g_cy, g_w, g_h)):
        d = loc_ref[0, i] - g
        a = jnp.abs(d)
        sl1 = jnp.where(a < 1.0, 0.5 * d * d, a - 0.5)
        acc = acc + jnp.where(pos, sl1, 0.0)
    loss_l_ref[0, 0, 0] = jnp.sum(acc)


def _conf_kernel(conf_ref, conf_t_ref, ce_ref):
    x = conf_ref[...]
    num = x.shape[0]
    lse = jnp.log(jnp.sum(jnp.exp(x), axis=2))
    cls = jax.lax.broadcasted_iota(jnp.int32, (num, _TP, _NUM_CLASSES), 2)
    tt = conf_t_ref[...][:, :, None]
    gathered = jnp.sum(jnp.where(cls == tt, x, 0.0), axis=2)
    ce_ref[...] = lse - gathered


def _mine_kernel(ce_raw_ref, conf_t_ref, num_pos_ref, loss_l_ref,
                 out_l_ref, out_c_ref):
    ce_raw = ce_raw_ref[...]
    num = ce_raw.shape[0]
    col = jax.lax.broadcasted_iota(jnp.int32, (num, _PPAD), 1)
    pos = conf_t_ref[...] > 0
    pos_sum = jnp.sum(jnp.where(pos, ce_raw, 0.0), axis=1, keepdims=True)
    ce = jnp.where(pos | (col >= _P), 0.0, ce_raw)
    np_i = num_pos_ref[...]
    k = jnp.minimum(np_i * _NEGPOS_RATIO, _P - 1).astype(jnp.float32)

    def body(i, tbits):
        bit = jnp.left_shift(jnp.int32(1), 30 - i)
        trial = jnp.bitwise_or(tbits, bit)
        trialf = jax.lax.bitcast_convert_type(trial, jnp.float32)
        cnt = jnp.sum((ce >= trialf).astype(jnp.float32), axis=1,
                      keepdims=True)
        return jnp.where(cnt >= k, trial, tbits)

    tbits = jax.lax.fori_loop(0, 31, body,
                              jnp.zeros((num, 1), dtype=jnp.int32))
    thr = jax.lax.bitcast_convert_type(tbits, jnp.float32)
    gt = ce > thr
    sum_gt = jnp.sum(jnp.where(gt, ce, 0.0), axis=1, keepdims=True)
    cnt_gt = jnp.sum(gt.astype(jnp.float32), axis=1, keepdims=True)
    top_sum = sum_gt + (k - cnt_gt) * thr
    loss_c = jnp.sum(pos_sum + top_sum)
    n = jnp.sum(np_i).astype(jnp.float32)
    out_l_ref[0, 0] = jnp.sum(loss_l_ref[...]) / n
    out_c_ref[0, 0] = loss_c / n


def kernel(loc_data, conf_data, priors, boxes, labels):
    num = conf_data.shape[0]
    priors_p = jnp.pad(priors, ((0, _PPAD - _P), (0, 0))).T.reshape(
        4, _ROWS, _LANES)
    loc_p = jnp.pad(loc_data, ((0, 0), (0, _PPAD - _P), (0, 0))).transpose(
        0, 2, 1).reshape(num, 4, _ROWS, _LANES)
    labels3 = labels.reshape(num, 1, _NOBJ)

    conf_t, num_pos, loss_l_b = pl.pallas_call(
        _match_kernel,
        grid=(num,),
        in_specs=[
            pl.BlockSpec((4, _ROWS, _LANES), lambda b: (0, 0, 0)),
            pl.BlockSpec((1, _NOBJ, 4), lambda b: (b, 0, 0),
                         memory_space=pltpu.SMEM),
            pl.BlockSpec((1, 1, _NOBJ), lambda b: (b, 0, 0),
                         memory_space=pltpu.SMEM),
            pl.BlockSpec((1, 4, _ROWS, _LANES), lambda b: (b, 0, 0, 0)),
        ],
        out_specs=[
            pl.BlockSpec((1, _ROWS, _LANES), lambda b: (b, 0, 0)),
            pl.BlockSpec((1, 1, 1), lambda b: (b, 0, 0),
                         memory_space=pltpu.SMEM),
            pl.BlockSpec((1, 1, 1), lambda b: (b, 0, 0),
                         memory_space=pltpu.SMEM),
        ],
        out_shape=[
            jax.ShapeDtypeStruct((num, _ROWS, _LANES), jnp.int32),
            jax.ShapeDtypeStruct((num, 1, 1), jnp.int32),
            jax.ShapeDtypeStruct((num, 1, 1), jnp.float32),
        ],
    )(priors_p, boxes, labels3, loc_p)

    conf_t2 = conf_t.reshape(num, _PPAD)

    ce = pl.pallas_call(
        _conf_kernel,
        grid=(_NT,),
        in_specs=[
            pl.BlockSpec((num, _TP, _NUM_CLASSES), lambda j: (0, j, 0)),
            pl.BlockSpec((num, _TP), lambda j: (0, j)),
        ],
        out_specs=pl.BlockSpec((num, _TP), lambda j: (0, j)),
        out_shape=jax.ShapeDtypeStruct((num, _PPAD), jnp.float32),
    )(conf_data, conf_t2)

    out_l, out_c = pl.pallas_call(
        _mine_kernel,
        in_specs=[
            pl.BlockSpec(memory_space=pltpu.VMEM),
            pl.BlockSpec(memory_space=pltpu.VMEM),
            pl.BlockSpec(memory_space=pltpu.VMEM),
            pl.BlockSpec(memory_space=pltpu.VMEM),
        ],
        out_specs=[
            pl.BlockSpec(memory_space=pltpu.SMEM),
            pl.BlockSpec(memory_space=pltpu.SMEM),
        ],
        out_shape=[
            jax.ShapeDtypeStruct((1, 1), jnp.float32),
            jax.ShapeDtypeStruct((1, 1), jnp.float32),
        ],
    )(ce, conf_t2, num_pos.reshape(num, 1), loss_l_b.reshape(num, 1))

    return out_l.reshape(()), out_c.reshape(())
